# baseline (speedup 1.0000x reference)
import functools

import jax
import jax.numpy as jnp
from jax import lax
from jax.experimental import pallas as pl
from jax.experimental.pallas import tpu as pltpu

B, T, N, F_IN = 8, 12, 1000, 2
HEADS, C = 8, 8
HID_GAT = HEADS * C
HID_LSTM = 128
N_EDGES = 16000
BN = B * N
D = N * HID_GAT
G4 = 4 * HID_LSTM


def _prep_body(x_ref, wg_ref, asv_ref, adv_ref, bg_ref,
               seqd_ref, h0c_ref, as_ref, ad_ref):
    xt = x_ref[:, 0].reshape(BN, F_IN)
    w = wg_ref[:]
    x0 = xt[:, 0:1]
    x1 = xt[:, 1:2]
    h = x0 * w[0:1, :] + x1 * w[1:2, :]
    hb = h + bg_ref[:]
    seqd_ref[0] = jnp.where(hb > 0, hb, jnp.exp(jnp.minimum(hb, 0.0)) - 1.0)
    h0 = h[:N, :]
    h0c_ref[0] = h0
    as_ref[0] = (h0 * asv_ref[:]).reshape(N, HEADS, C).sum(-1)
    ad_ref[0] = (h0 * adv_ref[:]).reshape(N, HEADS, C).sum(-1)


def _prep(x, W_gat, att_src, att_dst, b_gat):
    return pl.pallas_call(
        _prep_body,
        grid=(T,),
        in_specs=[
            pl.BlockSpec((B, 1, N, F_IN), lambda t: (0, t, 0, 0)),
            pl.BlockSpec((F_IN, HID_GAT), lambda t: (0, 0)),
            pl.BlockSpec((1, HID_GAT), lambda t: (0, 0)),
            pl.BlockSpec((1, HID_GAT), lambda t: (0, 0)),
            pl.BlockSpec((1, HID_GAT), lambda t: (0, 0)),
        ],
        out_specs=[
            pl.BlockSpec((1, BN, HID_GAT), lambda t: (t, 0, 0)),
            pl.BlockSpec((1, N, HID_GAT), lambda t: (t, 0, 0)),
            pl.BlockSpec((1, N, HEADS), lambda t: (t, 0, 0)),
            pl.BlockSpec((1, N, HEADS), lambda t: (t, 0, 0)),
        ],
        out_shape=[
            jax.ShapeDtypeStruct((T, BN, HID_GAT), jnp.float32),
            jax.ShapeDtypeStruct((T, N, HID_GAT), jnp.float32),
            jax.ShapeDtypeStruct((T, N, HEADS), jnp.float32),
            jax.ShapeDtypeStruct((T, N, HEADS), jnp.float32),
        ],
    )(x, W_gat, att_src.reshape(1, HID_GAT), att_dst.reshape(1, HID_GAT),
      b_gat.reshape(1, HID_GAT))


NP = 1024
E_AUG = 17408
ECH = 128
HH = HEADS // 2
CW = HH * C
HS = HH + 1
H0S = CW
MW = 40


def _sc_edges(src1d, dst1d, a4s, a4d, h0c, bg):
    from jax.experimental.pallas import tpu_sc as plsc

    mesh = plsc.VectorSubcoreMesh(core_axis_name="c", subcore_axis_name="s")

    @functools.partial(
        pl.kernel, mesh=mesh,
        out_type=jax.ShapeDtypeStruct((T, 2, NP * CW), jnp.float32),
        scratch_types=[
            pltpu.VMEM((E_AUG,), jnp.int32),
            pltpu.VMEM((E_AUG,), jnp.int32),
            pltpu.VMEM((NP * HS,), jnp.float32),
            pltpu.VMEM((NP * HS,), jnp.float32),
            pltpu.VMEM((NP * H0S,), jnp.float32),
            pltpu.VMEM((NP * MW,), jnp.float32),
            pltpu.VMEM((ECH * CW,), jnp.float32),
            pltpu.VMEM((HID_GAT,), jnp.float32),
        ],
        compiler_params=pltpu.CompilerParams(needs_layout_passes=False),
    )
    def k(src_hbm, dst_hbm, a4s_hbm, a4d_hbm, h0c_hbm, bg_hbm, out_hbm,
          src_v, dst_v, as_v, ad_v, h0_v, acc_v, outc_v, bg_v):
        cid = lax.axis_index("c")
        sid = lax.axis_index("s")

        @pl.when(sid < 12)
        def _():
            th = sid // 2
            t = cid * 6 + th
            hh = sid - 2 * th
            lane = lax.iota(jnp.int32, 16)
            zero16 = jnp.zeros((16,), jnp.float32)

            pltpu.sync_copy(src_hbm, src_v)
            pltpu.sync_copy(dst_hbm, dst_v)
            pltpu.sync_copy(a4s_hbm.at[t, hh], as_v)
            pltpu.sync_copy(a4d_hbm.at[t, hh], ad_v)
            pltpu.sync_copy(h0c_hbm.at[t, hh], h0_v)
            pltpu.sync_copy(bg_hbm, bg_v)

            @plsc.parallel_loop(0, NP * MW // 16)
            def _z_body(i):
                acc_v[pl.ds(i * 16, 16)] = zero16

            lo_mask = lane < 8

            @plsc.parallel_loop(0, E_AUG // 16, unroll=4)
            def _grp(g):
                src16 = src_v[pl.ds(g * 16, 16)]
                dst16 = dst_v[pl.ds(g * 16, 16)]
                src4 = src16 * HS
                dst4 = dst16 * HS
                dst40 = dst16 * MW
                exs = []
                for hp in range(HH):
                    av = plsc.load_gather(as_v, [src4 + hp])
                    bv = plsc.load_gather(ad_v, [dst4 + hp])
                    e = av + bv
                    e = jnp.maximum(e, 0.2 * e)
                    ex = jnp.exp(e)
                    plsc.addupdate_scatter(acc_v, [dst40 + (CW + hp)], ex)
                    exs.append(ex)
                for jj in range(16):
                    sb = src16[jj] * H0S
                    db = dst16[jj] * MW
                    hlo = h0_v[pl.ds(sb, 16)]
                    hhi = h0_v[pl.ds(sb + 16, 16)]
                    mlo = jnp.where(lo_mask, hlo * exs[0][jj],
                                    hlo * exs[1][jj])
                    mhi = jnp.where(lo_mask, hhi * exs[2][jj],
                                    hhi * exs[3][jj])
                    plsc.addupdate(acc_v.at[pl.ds(db, 16)], mlo)
                    plsc.addupdate(acc_v.at[pl.ds(db + 16, 16)], mhi)

            bglo = bg_v[pl.ds(hh * CW, 16)]
            bghi = bg_v[pl.ds(hh * CW + 16, 16)]

            def _elu(o):
                return jnp.where(o > 0.0, o,
                                 jnp.exp(jnp.minimum(o, 0.0)) - 1.0)

            def _fin(fc, _):
                @plsc.parallel_loop(0, ECH)
                def _fn(nl):
                    nb = (fc * ECH + nl) * MW
                    vlo = acc_v[pl.ds(nb, 16)]
                    vhi = acc_v[pl.ds(nb + 16, 16)]
                    sv = acc_v[pl.ds(nb + 24, 16)]
                    rec = 1.0 / (sv + 1e-16)
                    olo = jnp.where(lo_mask, vlo * rec[8], vlo * rec[9])
                    ohi = jnp.where(lo_mask, vhi * rec[10], vhi * rec[11])
                    outc_v[pl.ds(nl * CW, 16)] = _elu(olo + bglo)
                    outc_v[pl.ds(nl * CW + 16, 16)] = _elu(ohi + bghi)
                pltpu.sync_copy(
                    outc_v,
                    out_hbm.at[t, hh, pl.ds(fc * ECH * CW, ECH * CW)])
                return 0
            lax.fori_loop(0, NP // ECH, _fin, 0)

    return k(src1d, dst1d, a4s, a4d, h0c, bg)


def _lstm_body(a_ref, lo_ref, hi_ref, w_ref, whh_ref, bih_ref, bhh_ref,
               wfc_ref, bfc_ref, out_ref, acc_ref, *, nk):
    k = pl.program_id(0)

    @pl.when(k == 0)
    def _():
        acc_ref[:] = jnp.zeros_like(acc_ref)

    a = a_ref[:]
    nblk = a.shape[1] // HID_GAT
    o3 = jnp.concatenate(
        [lo_ref[:].reshape(T, nblk, CW), hi_ref[:].reshape(T, nblk, CW)],
        axis=2).reshape(T, a.shape[1])
    rep = jnp.broadcast_to(o3[:, None, :],
                           (T, B, a.shape[1])).reshape(T * B, a.shape[1])
    row = lax.broadcasted_iota(jnp.int32, a.shape, 0)
    a = jnp.where(row % B == 0, rep, a)
    acc_ref[:] += lax.dot_general(
        a, w_ref[:], (((1,), (1,)), ((), ())),
        preferred_element_type=jnp.float32)

    @pl.when(k == nk - 1)
    def _():
        whh = whh_ref[:]
        bias = (bih_ref[:] + bhh_ref[:])

        def step(t, carry):
            h, c = carry
            row = pl.multiple_of(t * B, 8)
            gates = acc_ref[pl.ds(row, B), :] + lax.dot_general(
                h, whh, (((1,), (1,)), ((), ())),
                preferred_element_type=jnp.float32) + bias
            i = jax.nn.sigmoid(gates[:, 0:HID_LSTM])
            f = jax.nn.sigmoid(gates[:, HID_LSTM:2 * HID_LSTM])
            g = jnp.tanh(gates[:, 2 * HID_LSTM:3 * HID_LSTM])
            o = jax.nn.sigmoid(gates[:, 3 * HID_LSTM:4 * HID_LSTM])
            c = f * c + i * g
            h = o * jnp.tanh(c)
            return (h, c)

        h0 = jnp.zeros((B, HID_LSTM), jnp.float32)
        c0 = jnp.zeros((B, HID_LSTM), jnp.float32)
        h, _ = lax.fori_loop(0, T, step, (h0, c0))
        out_ref[:] = lax.dot_general(
            h, wfc_ref[:], (((1,), (1,)), ((), ())),
            preferred_element_type=jnp.float32) + bfc_ref[:]


def _lstm_fc(seq2d, out0_lo, out0_hi, W_ih, W_hh, b_ih, b_hh, W_fc, b_fc):
    KB = 6400
    nk = D // KB
    return pl.pallas_call(
        functools.partial(_lstm_body, nk=nk),
        grid=(nk,),
        in_specs=[
            pl.BlockSpec((T * B, KB), lambda k: (0, k)),
            pl.BlockSpec((T, KB // 2), lambda k: (0, k)),
            pl.BlockSpec((T, KB // 2), lambda k: (0, k)),
            pl.BlockSpec((G4, KB), lambda k: (0, k)),
            pl.BlockSpec((G4, HID_LSTM), lambda k: (0, 0)),
            pl.BlockSpec((1, G4), lambda k: (0, 0)),
            pl.BlockSpec((1, G4), lambda k: (0, 0)),
            pl.BlockSpec((N * F_IN, HID_LSTM), lambda k: (0, 0)),
            pl.BlockSpec((1, N * F_IN), lambda k: (0, 0)),
        ],
        out_specs=pl.BlockSpec((B, N * F_IN), lambda k: (0, 0)),
        out_shape=jax.ShapeDtypeStruct((B, N * F_IN), jnp.float32),
        scratch_shapes=[pltpu.VMEM((T * B, G4), jnp.float32)],
    )(seq2d, out0_lo, out0_hi, W_ih, W_hh,
      b_ih.reshape(1, G4), b_hh.reshape(1, G4),
      W_fc, b_fc.reshape(1, N * F_IN))


def kernel(x, W_gat, att_src, att_dst, b_gat, W_ih, W_hh, b_ih, b_hh,
           W_fc, b_fc, edge_index):
    seqd, h0, a_src, a_dst = _prep(x, W_gat, att_src, att_dst, b_gat)

    n_pad = E_AUG - N_EDGES - N
    loop = jnp.arange(N, dtype=jnp.int32)
    padv = (N + jnp.arange(n_pad, dtype=jnp.int32) % (NP - N))
    src1d = jnp.concatenate([edge_index[0], loop, padv])
    dst1d = jnp.concatenate([edge_index[1], loop, padv])

    def _half_split(a, w, ws):
        a = jnp.pad(a, ((0, 0), (0, NP - N), (0, 0)))
        a = a.reshape(T, NP, 2, w).transpose(0, 2, 1, 3)
        a = jnp.pad(a, ((0, 0), (0, 0), (0, 0), (0, ws - w)))
        return a.reshape(T, 2, NP * ws)

    a4s = _half_split(a_src, HH, HS)
    a4d = _half_split(a_dst, HH, HS)
    h0c = _half_split(h0, CW, H0S)
    out0p = _sc_edges(src1d, dst1d, a4s, a4d, h0c, b_gat)
    o2 = out0p
    seq2d = seqd.reshape(T * B, D)
    out = _lstm_fc(seq2d, o2[:, 0], o2[:, 1], W_ih, W_hh, b_ih, b_hh,
                   W_fc, b_fc)
    return out.reshape(B, N, F_IN)


# speedup vs baseline: 1.0931x; 1.0931x over previous
import functools

import jax
import jax.numpy as jnp
from jax import lax
from jax.experimental import pallas as pl
from jax.experimental.pallas import tpu as pltpu

B, T, N, F_IN = 8, 12, 1000, 2
HEADS, C = 8, 8
HID_GAT = HEADS * C
HID_LSTM = 128
N_EDGES = 16000
BN = B * N
D = N * HID_GAT
G4 = 4 * HID_LSTM


def _prep_body(x_ref, wg_ref, asv_ref, adv_ref, bg_ref,
               seqd_ref, h0c_ref, as_ref, ad_ref):
    xt = x_ref[:, 0].reshape(BN, F_IN)
    w = wg_ref[:]
    x0 = xt[:, 0:1]
    x1 = xt[:, 1:2]
    h = x0 * w[0:1, :] + x1 * w[1:2, :]
    hb = h + bg_ref[:]
    seqd_ref[0] = jnp.where(hb > 0, hb, jnp.exp(jnp.minimum(hb, 0.0)) - 1.0)
    h0 = h[:N, :]
    zrows = jnp.zeros((NP - N, CW), jnp.float32)
    h0c_ref[0] = jnp.concatenate(
        [h0[:, :CW], zrows, h0[:, CW:], zrows])
    as_ref[0] = (h0 * asv_ref[:]).reshape(N, HEADS, C).sum(-1)
    ad_ref[0] = (h0 * adv_ref[:]).reshape(N, HEADS, C).sum(-1)


def _prep(x, W_gat, att_src, att_dst, b_gat):
    return pl.pallas_call(
        _prep_body,
        grid=(T,),
        in_specs=[
            pl.BlockSpec((B, 1, N, F_IN), lambda t: (0, t, 0, 0)),
            pl.BlockSpec((F_IN, HID_GAT), lambda t: (0, 0)),
            pl.BlockSpec((1, HID_GAT), lambda t: (0, 0)),
            pl.BlockSpec((1, HID_GAT), lambda t: (0, 0)),
            pl.BlockSpec((1, HID_GAT), lambda t: (0, 0)),
        ],
        out_specs=[
            pl.BlockSpec((1, BN, HID_GAT), lambda t: (t, 0, 0)),
            pl.BlockSpec((1, 2 * NP, CW), lambda t: (t, 0, 0)),
            pl.BlockSpec((1, N, HEADS), lambda t: (t, 0, 0)),
            pl.BlockSpec((1, N, HEADS), lambda t: (t, 0, 0)),
        ],
        out_shape=[
            jax.ShapeDtypeStruct((T, BN, HID_GAT), jnp.float32),
            jax.ShapeDtypeStruct((T, 2 * NP, CW), jnp.float32),
            jax.ShapeDtypeStruct((T, N, HEADS), jnp.float32),
            jax.ShapeDtypeStruct((T, N, HEADS), jnp.float32),
        ],
    )(x, W_gat, att_src.reshape(1, HID_GAT), att_dst.reshape(1, HID_GAT),
      b_gat.reshape(1, HID_GAT))


NP = 1024
E_AUG = 17408
ECH = 128
HH = HEADS // 2
CW = HH * C
HS = HH + 1
H0S = CW
MW = 40


def _sc_edges(src1d, dst1d, a4s, a4d, h0c, bg):
    from jax.experimental.pallas import tpu_sc as plsc

    mesh = plsc.VectorSubcoreMesh(core_axis_name="c", subcore_axis_name="s")

    @functools.partial(
        pl.kernel, mesh=mesh,
        out_type=jax.ShapeDtypeStruct((T, 2, NP * CW), jnp.float32),
        scratch_types=[
            pltpu.VMEM((E_AUG,), jnp.int32),
            pltpu.VMEM((E_AUG,), jnp.int32),
            pltpu.VMEM((NP * HS,), jnp.float32),
            pltpu.VMEM((NP * HS,), jnp.float32),
            pltpu.VMEM((NP * H0S,), jnp.float32),
            pltpu.VMEM((NP * MW,), jnp.float32),
            pltpu.VMEM((ECH * CW,), jnp.float32),
            pltpu.VMEM((HID_GAT,), jnp.float32),
        ],
        compiler_params=pltpu.CompilerParams(needs_layout_passes=False),
    )
    def k(src_hbm, dst_hbm, a4s_hbm, a4d_hbm, h0c_hbm, bg_hbm, out_hbm,
          src_v, dst_v, as_v, ad_v, h0_v, acc_v, outc_v, bg_v):
        cid = lax.axis_index("c")
        sid = lax.axis_index("s")

        @pl.when(sid < 12)
        def _():
            th = sid // 2
            t = cid * 6 + th
            hh = sid - 2 * th
            lane = lax.iota(jnp.int32, 16)
            zero16 = jnp.zeros((16,), jnp.float32)

            pltpu.sync_copy(src_hbm, src_v)
            pltpu.sync_copy(dst_hbm, dst_v)
            pltpu.sync_copy(a4s_hbm.at[t, hh], as_v)
            pltpu.sync_copy(a4d_hbm.at[t, hh], ad_v)
            pltpu.sync_copy(h0c_hbm.at[t, hh], h0_v)
            pltpu.sync_copy(bg_hbm, bg_v)

            @plsc.parallel_loop(0, NP * MW // 16)
            def _z_body(i):
                acc_v[pl.ds(i * 16, 16)] = zero16

            lo_mask = lane < 8

            @plsc.parallel_loop(0, E_AUG // 16, unroll=2)
            def _grp(g):
                src16 = src_v[pl.ds(g * 16, 16)]
                dst16 = dst_v[pl.ds(g * 16, 16)]
                src4 = src16 * HS
                dst4 = dst16 * HS
                dst40 = dst16 * MW
                exs = []
                for hp in range(HH):
                    av = plsc.load_gather(as_v, [src4 + hp])
                    bv = plsc.load_gather(ad_v, [dst4 + hp])
                    e = av + bv
                    e = jnp.maximum(e, 0.2 * e)
                    ex = jnp.exp(e)
                    plsc.addupdate_scatter(acc_v, [dst40 + (CW + hp)], ex)
                    exs.append(ex)
                for jj in range(16):
                    sb = src16[jj] * H0S
                    db = dst16[jj] * MW
                    hlo = h0_v[pl.ds(sb, 16)]
                    hhi = h0_v[pl.ds(sb + 16, 16)]
                    mlo = jnp.where(lo_mask, hlo * exs[0][jj],
                                    hlo * exs[1][jj])
                    mhi = jnp.where(lo_mask, hhi * exs[2][jj],
                                    hhi * exs[3][jj])
                    plsc.addupdate(acc_v.at[pl.ds(db, 16)], mlo)
                    plsc.addupdate(acc_v.at[pl.ds(db + 16, 16)], mhi)

            bglo = bg_v[pl.ds(hh * CW, 16)]
            bghi = bg_v[pl.ds(hh * CW + 16, 16)]

            def _elu(o):
                return jnp.where(o > 0.0, o,
                                 jnp.exp(jnp.minimum(o, 0.0)) - 1.0)

            def _fin(fc, _):
                @plsc.parallel_loop(0, ECH)
                def _fn(nl):
                    nb = (fc * ECH + nl) * MW
                    vlo = acc_v[pl.ds(nb, 16)]
                    vhi = acc_v[pl.ds(nb + 16, 16)]
                    sv = acc_v[pl.ds(nb + 24, 16)]
                    rec = 1.0 / (sv + 1e-16)
                    olo = jnp.where(lo_mask, vlo * rec[8], vlo * rec[9])
                    ohi = jnp.where(lo_mask, vhi * rec[10], vhi * rec[11])
                    outc_v[pl.ds(nl * CW, 16)] = _elu(olo + bglo)
                    outc_v[pl.ds(nl * CW + 16, 16)] = _elu(ohi + bghi)
                pltpu.sync_copy(
                    outc_v,
                    out_hbm.at[t, hh, pl.ds(fc * ECH * CW, ECH * CW)])
                return 0
            lax.fori_loop(0, NP // ECH, _fin, 0)

    return k(src1d, dst1d, a4s, a4d, h0c, bg)


def _lstm_body(a_ref, lo_ref, hi_ref, w_ref, whh_ref, bih_ref, bhh_ref,
               wfc_ref, bfc_ref, out_ref, acc_ref, *, nk):
    k = pl.program_id(0)

    @pl.when(k == 0)
    def _():
        acc_ref[:] = jnp.zeros_like(acc_ref)

    a = a_ref[:]
    nblk = a.shape[1] // HID_GAT
    o3 = jnp.concatenate(
        [lo_ref[:].reshape(T, nblk, CW), hi_ref[:].reshape(T, nblk, CW)],
        axis=2).reshape(T, a.shape[1])
    rep = jnp.broadcast_to(o3[:, None, :],
                           (T, B, a.shape[1])).reshape(T * B, a.shape[1])
    row = lax.broadcasted_iota(jnp.int32, a.shape, 0)
    a = jnp.where(row % B == 0, rep, a)
    acc_ref[:] += lax.dot_general(
        a, w_ref[:], (((1,), (1,)), ((), ())),
        preferred_element_type=jnp.float32)

    @pl.when(k == nk - 1)
    def _():
        whh = whh_ref[:]
        bias = (bih_ref[:] + bhh_ref[:])

        def step(t, carry):
            h, c = carry
            row = pl.multiple_of(t * B, 8)
            gates = acc_ref[pl.ds(row, B), :] + lax.dot_general(
                h, whh, (((1,), (1,)), ((), ())),
                preferred_element_type=jnp.float32) + bias
            i = jax.nn.sigmoid(gates[:, 0:HID_LSTM])
            f = jax.nn.sigmoid(gates[:, HID_LSTM:2 * HID_LSTM])
            g = jnp.tanh(gates[:, 2 * HID_LSTM:3 * HID_LSTM])
            o = jax.nn.sigmoid(gates[:, 3 * HID_LSTM:4 * HID_LSTM])
            c = f * c + i * g
            h = o * jnp.tanh(c)
            return (h, c)

        h0 = jnp.zeros((B, HID_LSTM), jnp.float32)
        c0 = jnp.zeros((B, HID_LSTM), jnp.float32)
        h, _ = lax.fori_loop(0, T, step, (h0, c0))
        out_ref[:] = lax.dot_general(
            h, wfc_ref[:], (((1,), (1,)), ((), ())),
            preferred_element_type=jnp.float32) + bfc_ref[:]


def _lstm_fc(seq2d, out0_lo, out0_hi, W_ih, W_hh, b_ih, b_hh, W_fc, b_fc):
    KB = 6400
    nk = D // KB
    return pl.pallas_call(
        functools.partial(_lstm_body, nk=nk),
        grid=(nk,),
        in_specs=[
            pl.BlockSpec((T * B, KB), lambda k: (0, k)),
            pl.BlockSpec((T, KB // 2), lambda k: (0, k)),
            pl.BlockSpec((T, KB // 2), lambda k: (0, k)),
            pl.BlockSpec((G4, KB), lambda k: (0, k)),
            pl.BlockSpec((G4, HID_LSTM), lambda k: (0, 0)),
            pl.BlockSpec((1, G4), lambda k: (0, 0)),
            pl.BlockSpec((1, G4), lambda k: (0, 0)),
            pl.BlockSpec((N * F_IN, HID_LSTM), lambda k: (0, 0)),
            pl.BlockSpec((1, N * F_IN), lambda k: (0, 0)),
        ],
        out_specs=pl.BlockSpec((B, N * F_IN), lambda k: (0, 0)),
        out_shape=jax.ShapeDtypeStruct((B, N * F_IN), jnp.float32),
        scratch_shapes=[pltpu.VMEM((T * B, G4), jnp.float32)],
    )(seq2d, out0_lo, out0_hi, W_ih, W_hh,
      b_ih.reshape(1, G4), b_hh.reshape(1, G4),
      W_fc, b_fc.reshape(1, N * F_IN))


def kernel(x, W_gat, att_src, att_dst, b_gat, W_ih, W_hh, b_ih, b_hh,
           W_fc, b_fc, edge_index):
    seqd, h0c, a_src, a_dst = _prep(x, W_gat, att_src, att_dst, b_gat)

    n_pad = E_AUG - N_EDGES - N
    loop = jnp.arange(N, dtype=jnp.int32)
    padv = (N + jnp.arange(n_pad, dtype=jnp.int32) % (NP - N))
    src1d = jnp.concatenate([edge_index[0], loop, padv])
    dst1d = jnp.concatenate([edge_index[1], loop, padv])

    def _half_split(a, w, ws):
        a = jnp.pad(a, ((0, 0), (0, NP - N), (0, 0)))
        a = a.reshape(T, NP, 2, w).transpose(0, 2, 1, 3)
        a = jnp.pad(a, ((0, 0), (0, 0), (0, 0), (0, ws - w)))
        return a.reshape(T, 2, NP * ws)

    a4s = _half_split(a_src, HH, HS)
    a4d = _half_split(a_dst, HH, HS)
    out0p = _sc_edges(src1d, dst1d, a4s, a4d,
                      h0c.reshape(T, 2, NP * CW), b_gat)
    o2 = out0p
    seq2d = seqd.reshape(T * B, D)
    out = _lstm_fc(seq2d, o2[:, 0], o2[:, 1], W_ih, W_hh, b_ih, b_hh,
                   W_fc, b_fc)
    return out.reshape(B, N, F_IN)


# speedup vs baseline: 1.1313x; 1.0350x over previous
import functools

import jax
import jax.numpy as jnp
from jax import lax
from jax.experimental import pallas as pl
from jax.experimental.pallas import tpu as pltpu

B, T, N, F_IN = 8, 12, 1000, 2
HEADS, C = 8, 8
HID_GAT = HEADS * C
HID_LSTM = 128
N_EDGES = 16000
BN = B * N
D = N * HID_GAT
G4 = 4 * HID_LSTM


def _prep_body(x_ref, wg_ref, asv_ref, adv_ref, bg_ref,
               seqd_ref, h0c_ref, as_ref, ad_ref):
    xt = x_ref[:, 0].reshape(BN, F_IN)
    w = wg_ref[:]
    x0 = xt[:, 0:1]
    x1 = xt[:, 1:2]
    h = x0 * w[0:1, :] + x1 * w[1:2, :]
    hb = h + bg_ref[:]
    seqd_ref[0] = jnp.where(hb > 0, hb, jnp.exp(jnp.minimum(hb, 0.0)) - 1.0)
    h0 = h[:N, :]
    zrows = jnp.zeros((NP - N, CW), jnp.float32)
    h0c_ref[0] = jnp.concatenate(
        [h0[:, :CW], zrows, h0[:, CW:], zrows])
    asrc = (h0 * asv_ref[:]).reshape(N, HEADS, C).sum(-1)
    adst = (h0 * adv_ref[:]).reshape(N, HEADS, C).sum(-1)
    zcol = jnp.zeros((N, 1), jnp.float32)
    zrows5 = jnp.zeros((NP - N, HS), jnp.float32)

    def _a5(a):
        return jnp.concatenate(
            [jnp.concatenate([a[:, :HH], zcol], axis=1), zrows5,
             jnp.concatenate([a[:, HH:], zcol], axis=1), zrows5])

    as_ref[0] = _a5(asrc)
    ad_ref[0] = _a5(adst)


def _prep(x, W_gat, att_src, att_dst, b_gat):
    return pl.pallas_call(
        _prep_body,
        grid=(T,),
        in_specs=[
            pl.BlockSpec((B, 1, N, F_IN), lambda t: (0, t, 0, 0)),
            pl.BlockSpec((F_IN, HID_GAT), lambda t: (0, 0)),
            pl.BlockSpec((1, HID_GAT), lambda t: (0, 0)),
            pl.BlockSpec((1, HID_GAT), lambda t: (0, 0)),
            pl.BlockSpec((1, HID_GAT), lambda t: (0, 0)),
        ],
        out_specs=[
            pl.BlockSpec((1, BN, HID_GAT), lambda t: (t, 0, 0)),
            pl.BlockSpec((1, 2 * NP, CW), lambda t: (t, 0, 0)),
            pl.BlockSpec((1, 2 * NP, HS), lambda t: (t, 0, 0)),
            pl.BlockSpec((1, 2 * NP, HS), lambda t: (t, 0, 0)),
        ],
        out_shape=[
            jax.ShapeDtypeStruct((T, BN, HID_GAT), jnp.float32),
            jax.ShapeDtypeStruct((T, 2 * NP, CW), jnp.float32),
            jax.ShapeDtypeStruct((T, 2 * NP, HS), jnp.float32),
            jax.ShapeDtypeStruct((T, 2 * NP, HS), jnp.float32),
        ],
    )(x, W_gat, att_src.reshape(1, HID_GAT), att_dst.reshape(1, HID_GAT),
      b_gat.reshape(1, HID_GAT))


NP = 1024
E_AUG = 17408
ECH = 128
HH = HEADS // 2
CW = HH * C
HS = HH + 1
H0S = CW
MW = 40


def _sc_edges(src1d, dst1d, a4s, a4d, h0c, bg):
    from jax.experimental.pallas import tpu_sc as plsc

    mesh = plsc.VectorSubcoreMesh(core_axis_name="c", subcore_axis_name="s")

    @functools.partial(
        pl.kernel, mesh=mesh,
        out_type=jax.ShapeDtypeStruct((T, 2, NP * CW), jnp.float32),
        scratch_types=[
            pltpu.VMEM((E_AUG,), jnp.int32),
            pltpu.VMEM((E_AUG,), jnp.int32),
            pltpu.VMEM((NP * HS,), jnp.float32),
            pltpu.VMEM((NP * HS,), jnp.float32),
            pltpu.VMEM((NP * H0S,), jnp.float32),
            pltpu.VMEM((NP * MW,), jnp.float32),
            pltpu.VMEM((ECH * CW,), jnp.float32),
            pltpu.VMEM((HID_GAT,), jnp.float32),
        ],
        compiler_params=pltpu.CompilerParams(needs_layout_passes=False),
    )
    def k(src_hbm, dst_hbm, a4s_hbm, a4d_hbm, h0c_hbm, bg_hbm, out_hbm,
          src_v, dst_v, as_v, ad_v, h0_v, acc_v, outc_v, bg_v):
        cid = lax.axis_index("c")
        sid = lax.axis_index("s")

        @pl.when(sid < 12)
        def _():
            th = sid // 2
            t = cid * 6 + th
            hh = sid - 2 * th
            lane = lax.iota(jnp.int32, 16)
            zero16 = jnp.zeros((16,), jnp.float32)

            pltpu.sync_copy(src_hbm, src_v)
            pltpu.sync_copy(dst_hbm, dst_v)
            pltpu.sync_copy(a4s_hbm.at[t, hh], as_v)
            pltpu.sync_copy(a4d_hbm.at[t, hh], ad_v)
            pltpu.sync_copy(h0c_hbm.at[t, hh], h0_v)
            pltpu.sync_copy(bg_hbm, bg_v)

            @plsc.parallel_loop(0, NP * MW // 16)
            def _z_body(i):
                acc_v[pl.ds(i * 16, 16)] = zero16

            lo_mask = lane < 8

            @plsc.parallel_loop(0, E_AUG // 16, unroll=2)
            def _grp(g):
                src16 = src_v[pl.ds(g * 16, 16)]
                dst16 = dst_v[pl.ds(g * 16, 16)]
                src4 = src16 * HS
                dst4 = dst16 * HS
                dst40 = dst16 * MW
                exs = []
                for hp in range(HH):
                    av = plsc.load_gather(as_v, [src4 + hp])
                    bv = plsc.load_gather(ad_v, [dst4 + hp])
                    e = av + bv
                    e = jnp.maximum(e, 0.2 * e)
                    ex = jnp.exp(e)
                    plsc.addupdate_scatter(acc_v, [dst40 + (CW + hp)], ex)
                    exs.append(ex)
                for jj in range(16):
                    sb = src16[jj] * H0S
                    db = dst16[jj] * MW
                    hlo = h0_v[pl.ds(sb, 16)]
                    hhi = h0_v[pl.ds(sb + 16, 16)]
                    mlo = jnp.where(lo_mask, hlo * exs[0][jj],
                                    hlo * exs[1][jj])
                    mhi = jnp.where(lo_mask, hhi * exs[2][jj],
                                    hhi * exs[3][jj])
                    plsc.addupdate(acc_v.at[pl.ds(db, 16)], mlo)
                    plsc.addupdate(acc_v.at[pl.ds(db + 16, 16)], mhi)

            bglo = bg_v[pl.ds(hh * CW, 16)]
            bghi = bg_v[pl.ds(hh * CW + 16, 16)]

            def _elu(o):
                return jnp.where(o > 0.0, o,
                                 jnp.exp(jnp.minimum(o, 0.0)) - 1.0)

            def _fin(fc, _):
                @plsc.parallel_loop(0, ECH)
                def _fn(nl):
                    nb = (fc * ECH + nl) * MW
                    vlo = acc_v[pl.ds(nb, 16)]
                    vhi = acc_v[pl.ds(nb + 16, 16)]
                    sv = acc_v[pl.ds(nb + 24, 16)]
                    rec = 1.0 / (sv + 1e-16)
                    olo = jnp.where(lo_mask, vlo * rec[8], vlo * rec[9])
                    ohi = jnp.where(lo_mask, vhi * rec[10], vhi * rec[11])
                    outc_v[pl.ds(nl * CW, 16)] = _elu(olo + bglo)
                    outc_v[pl.ds(nl * CW + 16, 16)] = _elu(ohi + bghi)
                pltpu.sync_copy(
                    outc_v,
                    out_hbm.at[t, hh, pl.ds(fc * ECH * CW, ECH * CW)])
                return 0
            lax.fori_loop(0, NP // ECH, _fin, 0)

    return k(src1d, dst1d, a4s, a4d, h0c, bg)


def _lstm_body(a_ref, lo_ref, hi_ref, w_ref, whh_ref, bih_ref, bhh_ref,
               wfc_ref, bfc_ref, out_ref, acc_ref, *, nk):
    k = pl.program_id(0)

    @pl.when(k == 0)
    def _():
        acc_ref[:] = jnp.zeros_like(acc_ref)

    a = a_ref[:]
    nblk = a.shape[1] // HID_GAT
    o3 = jnp.concatenate(
        [lo_ref[:].reshape(T, nblk, CW), hi_ref[:].reshape(T, nblk, CW)],
        axis=2).reshape(T, a.shape[1])
    rep = jnp.broadcast_to(o3[:, None, :],
                           (T, B, a.shape[1])).reshape(T * B, a.shape[1])
    row = lax.broadcasted_iota(jnp.int32, a.shape, 0)
    a = jnp.where(row % B == 0, rep, a)
    acc_ref[:] += lax.dot_general(
        a, w_ref[:], (((1,), (1,)), ((), ())),
        preferred_element_type=jnp.float32)

    @pl.when(k == nk - 1)
    def _():
        whh = whh_ref[:]
        bias = (bih_ref[:] + bhh_ref[:])

        def step(t, carry):
            h, c = carry
            row = pl.multiple_of(t * B, 8)
            gates = acc_ref[pl.ds(row, B), :] + lax.dot_general(
                h, whh, (((1,), (1,)), ((), ())),
                preferred_element_type=jnp.float32) + bias
            i = jax.nn.sigmoid(gates[:, 0:HID_LSTM])
            f = jax.nn.sigmoid(gates[:, HID_LSTM:2 * HID_LSTM])
            g = jnp.tanh(gates[:, 2 * HID_LSTM:3 * HID_LSTM])
            o = jax.nn.sigmoid(gates[:, 3 * HID_LSTM:4 * HID_LSTM])
            c = f * c + i * g
            h = o * jnp.tanh(c)
            return (h, c)

        h0 = jnp.zeros((B, HID_LSTM), jnp.float32)
        c0 = jnp.zeros((B, HID_LSTM), jnp.float32)
        h, _ = lax.fori_loop(0, T, step, (h0, c0))
        out_ref[:] = lax.dot_general(
            h, wfc_ref[:], (((1,), (1,)), ((), ())),
            preferred_element_type=jnp.float32) + bfc_ref[:]


def _lstm_fc(seq2d, out0_lo, out0_hi, W_ih, W_hh, b_ih, b_hh, W_fc, b_fc):
    KB = 6400
    nk = D // KB
    return pl.pallas_call(
        functools.partial(_lstm_body, nk=nk),
        grid=(nk,),
        in_specs=[
            pl.BlockSpec((T * B, KB), lambda k: (0, k)),
            pl.BlockSpec((T, KB // 2), lambda k: (0, k)),
            pl.BlockSpec((T, KB // 2), lambda k: (0, k)),
            pl.BlockSpec((G4, KB), lambda k: (0, k)),
            pl.BlockSpec((G4, HID_LSTM), lambda k: (0, 0)),
            pl.BlockSpec((1, G4), lambda k: (0, 0)),
            pl.BlockSpec((1, G4), lambda k: (0, 0)),
            pl.BlockSpec((N * F_IN, HID_LSTM), lambda k: (0, 0)),
            pl.BlockSpec((1, N * F_IN), lambda k: (0, 0)),
        ],
        out_specs=pl.BlockSpec((B, N * F_IN), lambda k: (0, 0)),
        out_shape=jax.ShapeDtypeStruct((B, N * F_IN), jnp.float32),
        scratch_shapes=[pltpu.VMEM((T * B, G4), jnp.float32)],
    )(seq2d, out0_lo, out0_hi, W_ih, W_hh,
      b_ih.reshape(1, G4), b_hh.reshape(1, G4),
      W_fc, b_fc.reshape(1, N * F_IN))


def kernel(x, W_gat, att_src, att_dst, b_gat, W_ih, W_hh, b_ih, b_hh,
           W_fc, b_fc, edge_index):
    seqd, h0c, a4s3, a4d3 = _prep(x, W_gat, att_src, att_dst, b_gat)

    n_pad = E_AUG - N_EDGES - N
    loop = jnp.arange(N, dtype=jnp.int32)
    padv = (N + jnp.arange(n_pad, dtype=jnp.int32) % (NP - N))
    src1d = jnp.concatenate([edge_index[0], loop, padv])
    dst1d = jnp.concatenate([edge_index[1], loop, padv])

    out0p = _sc_edges(src1d, dst1d,
                      a4s3.reshape(T, 2, NP * HS),
                      a4d3.reshape(T, 2, NP * HS),
                      h0c.reshape(T, 2, NP * CW), b_gat)
    o2 = out0p
    seq2d = seqd.reshape(T * B, D)
    out = _lstm_fc(seq2d, o2[:, 0], o2[:, 1], W_ih, W_hh, b_ih, b_hh,
                   W_fc, b_fc)
    return out.reshape(B, N, F_IN)
